# baseline (speedup 1.0000x reference)
import functools

import jax
import jax.numpy as jnp
from jax import lax
from jax.experimental import pallas as pl
from jax.experimental.pallas import tpu as pltpu
from jax.experimental.pallas import tpu_sc as plsc

NUM_V = 10000
NUM_E = 5000
NG = 16
MAXN = 1024
D = 128

NV_P = 10240
NE_P = 5120
NNZ_P = 327680
SENT_P = 16384
EMB_P = 12288
MP_P = 16384

CH = 128
NTILES = 32

_mesh = plsc.VectorSubcoreMesh(core_axis_name="c", subcore_axis_name="s")


def _wid():
    return lax.axis_index("c") * 16 + lax.axis_index("s")


def _gather_body(nchunks, src, idx, out, iv0, iv1, rb0, rb1, sem0, sem1):
    w = _wid()
    tbase = w * (nchunks * CH)
    ivs, rbs, sems = (iv0, iv1), (rb0, rb1), (sem0, sem1)

    def issue(t, b):
        pltpu.sync_copy(idx.at[pl.ds(tbase + t * CH, CH)], ivs[b])
        pltpu.async_copy(src.at[ivs[b]], rbs[b], sems[b])

    def finish(t, b):
        pltpu.make_async_copy(src.at[ivs[b]], rbs[b], sems[b]).wait()
        pltpu.sync_copy(rbs[b], out.at[pl.ds(tbase + t * CH, CH)])

    issue(0, 0)

    def pair(j, _):
        t0 = 2 * j

        @pl.when(t0 + 1 < nchunks)
        def _():
            issue(t0 + 1, 1)

        finish(t0, 0)

        @pl.when(t0 + 1 < nchunks)
        def _():
            @pl.when(t0 + 2 < nchunks)
            def _():
                issue(t0 + 2, 0)

            finish(t0 + 1, 1)

        return _

    lax.fori_loop(0, (nchunks + 1) // 2, pair, None)


def _sc_gather(src, idx, n_pad):
    nchunks = n_pad // (NTILES * CH)
    fn = pl.kernel(
        functools.partial(_gather_body, nchunks),
        out_type=jax.ShapeDtypeStruct((n_pad, D), jnp.float32),
        mesh=_mesh,
        scratch_types=[
            pltpu.VMEM((CH,), jnp.int32),
            pltpu.VMEM((CH,), jnp.int32),
            pltpu.VMEM((CH, D), jnp.float32),
            pltpu.VMEM((CH, D), jnp.float32),
            pltpu.SemaphoreType.DMA,
            pltpu.SemaphoreType.DMA,
        ],
    )
    return fn(src, idx)


def _k1_body(g_nchunks, c_nchunks, emb, xp, ridx, cidx,
             h_out, cv_out, ce_out,
             acc, iv0, iv1, iv2, iv3, rb0, rb1, sem0, sem1, semc):
    c = lax.axis_index("c")
    s = lax.axis_index("s")
    w = c * 16 + s
    ivs, rbs, sems = (iv0, iv1, iv2, iv3), (rb0, rb1), (sem0, sem1)

    tbase = w * (g_nchunks * CH)

    def g_issue(t, b):
        pltpu.sync_copy(xp.at[pl.ds(tbase + t * CH, CH)], ivs[b])
        pltpu.async_copy(emb.at[ivs[b]], rbs[b], sems[b])

    def g_finish(t, b):
        pltpu.make_async_copy(emb.at[ivs[b]], rbs[b], sems[b]).wait()
        pltpu.sync_copy(rbs[b], h_out.at[pl.ds(tbase + t * CH, CH)])

    g_issue(0, 0)

    def g_pair(j, _):
        t0 = 2 * j

        @pl.when(t0 + 1 < g_nchunks)
        def _():
            g_issue(t0 + 1, 1)

        g_finish(t0, 0)

        @pl.when(t0 + 1 < g_nchunks)
        def _():
            @pl.when(t0 + 2 < g_nchunks)
            def _():
                g_issue(t0 + 2, 0)

            g_finish(t0 + 1, 1)

        return _

    lax.fori_loop(0, (g_nchunks + 1) // 2, g_pair, None)

    def bfill(r, _):
        for cc in range(D // 16):
            rb0[r, pl.ds(cc * 16, 16)] = jnp.zeros((16,), jnp.float32)
            rb1[r, pl.ds(cc * 16, 16)] = jnp.ones((16,), jnp.float32)
        return _

    lax.fori_loop(0, CH, bfill, None)
    zsrc = rb0.at[pl.ds(0, 16)]

    def zacc_v(j, _):
        pltpu.sync_copy(zsrc, acc.at[pl.ds(s * (NV_P // 16) + j * 16, 16)])
        return _

    lax.fori_loop(0, NV_P // 16 // 16, zacc_v, None)
    plsc.subcore_barrier()

    def count_pass(idx_hbm):
        def group(g, _):
            for b in range(4):
                base = w * (c_nchunks * CH) + (g * 4 + b) * CH
                pltpu.sync_copy(idx_hbm.at[pl.ds(base, CH)], ivs[b])
                pltpu.async_copy(rb1, acc.at[ivs[b]], semc, add=True)
            for b in range(4):
                pltpu.make_async_copy(rb1, acc.at[ivs[b]], semc).wait()
            return _

        lax.fori_loop(0, c_nchunks // 4, group, None)

    count_pass(ridx)
    plsc.subcore_barrier()
    rv = NV_P // 16
    pltpu.sync_copy(acc.at[pl.ds(s * rv, rv)], cv_out.at[c, pl.ds(s * rv, rv)])
    plsc.subcore_barrier()

    def zacc_e(j, _):
        pltpu.sync_copy(zsrc, acc.at[pl.ds(s * (NE_P // 16) + j * 16, 16)])
        return _

    lax.fori_loop(0, NE_P // 16 // 16, zacc_e, None)
    plsc.subcore_barrier()
    count_pass(cidx)
    plsc.subcore_barrier()
    re = NE_P // 16
    pltpu.sync_copy(acc.at[pl.ds(s * re, re)], ce_out.at[c, pl.ds(s * re, re)])


def _sc_k1(emb, xp, ridx, cidx):
    g_nchunks = EMB_P // (NTILES * CH)
    c_nchunks = NNZ_P // (NTILES * CH)
    fn = pl.kernel(
        functools.partial(_k1_body, g_nchunks, c_nchunks),
        out_type=(
            jax.ShapeDtypeStruct((EMB_P, D), jnp.float32),
            jax.ShapeDtypeStruct((2, NV_P, D), jnp.float32),
            jax.ShapeDtypeStruct((2, NE_P, D), jnp.float32),
        ),
        mesh=_mesh,
        scratch_types=[
            pltpu.VMEM_SHARED((NV_P, D), jnp.float32),
            pltpu.VMEM((CH,), jnp.int32),
            pltpu.VMEM((CH,), jnp.int32),
            pltpu.VMEM((CH,), jnp.int32),
            pltpu.VMEM((CH,), jnp.int32),
            pltpu.VMEM((CH, D), jnp.float32),
            pltpu.VMEM((CH, D), jnp.float32),
            pltpu.SemaphoreType.DMA,
            pltpu.SemaphoreType.DMA,
            pltpu.SemaphoreType.DMA,
        ],
    )
    return fn(emb, xp, ridx, cidx)


def _mp_body(nchunks, src0, src1, idx, out0, out1,
             iv0, iv1, ra0, ra1, rc0, rc1, sem0, sem1):
    w = _wid()
    tbase = w * (nchunks * CH)
    ivs, ras, rcs, sems = (iv0, iv1), (ra0, ra1), (rc0, rc1), (sem0, sem1)

    def issue(t, b):
        pltpu.sync_copy(idx.at[pl.ds(tbase + t * CH, CH)], ivs[b])
        pltpu.async_copy(src0.at[ivs[b]], ras[b], sems[b])
        pltpu.async_copy(src1.at[ivs[b]], rcs[b], sems[b])

    def finish(t, b):
        pltpu.make_async_copy(src0.at[ivs[b]], ras[b], sems[b]).wait()
        pltpu.make_async_copy(src1.at[ivs[b]], rcs[b], sems[b]).wait()
        pltpu.sync_copy(ras[b], out0.at[pl.ds(tbase + t * CH, CH)])
        pltpu.sync_copy(rcs[b], out1.at[pl.ds(tbase + t * CH, CH)])

    issue(0, 0)

    def pair(j, _):
        t0 = 2 * j

        @pl.when(t0 + 1 < nchunks)
        def _():
            issue(t0 + 1, 1)

        finish(t0, 0)

        @pl.when(t0 + 1 < nchunks)
        def _():
            @pl.when(t0 + 2 < nchunks)
            def _():
                issue(t0 + 2, 0)

            finish(t0 + 1, 1)

        return _

    lax.fori_loop(0, (nchunks + 1) // 2, pair, None)


def _sc_mp(src0, src1, idx):
    nchunks = MP_P // (NTILES * CH)
    fn = pl.kernel(
        functools.partial(_mp_body, nchunks),
        out_type=(
            jax.ShapeDtypeStruct((MP_P, D), jnp.float32),
            jax.ShapeDtypeStruct((MP_P, D), jnp.float32),
        ),
        mesh=_mesh,
        scratch_types=[
            pltpu.VMEM((CH,), jnp.int32),
            pltpu.VMEM((CH,), jnp.int32),
            pltpu.VMEM((CH, D), jnp.float32),
            pltpu.VMEM((CH, D), jnp.float32),
            pltpu.VMEM((CH, D), jnp.float32),
            pltpu.VMEM((CH, D), jnp.float32),
            pltpu.SemaphoreType.DMA,
            pltpu.SemaphoreType.DMA,
        ],
    )
    return fn(src0, src1, idx)


def _spmm_body(nchunks, nout, src, gidx, sidx, vrep, outp,
               acc, gv0, gv1, sv0, sv1, vv0, vv1, rb0, rb1,
               sem0, sem1, sem2, sem3):
    c = lax.axis_index("c")
    s = lax.axis_index("s")
    w = c * 16 + s
    tbase = w * (nchunks * CH)
    gvs, svs, vvs = (gv0, gv1), (sv0, sv1), (vv0, vv1)
    rbs, sems = (rb0, rb1), (sem0, sem1, sem2, sem3)

    def zfill(r, _):
        for cc in range(D // 16):
            rb0[r, pl.ds(cc * 16, 16)] = jnp.zeros((16,), jnp.float32)
        return _

    lax.fori_loop(0, 16, zfill, None)
    rows_per_sub = nout // 16

    def zacc(j, _):
        pltpu.sync_copy(rb0.at[pl.ds(0, 16)],
                        acc.at[pl.ds(s * rows_per_sub + j * 16, 16)])
        return _

    lax.fori_loop(0, rows_per_sub // 16, zacc, None)
    plsc.subcore_barrier()

    vbase = w * (nchunks * (CH // 8))

    def issue(t, b):
        base = tbase + t * CH
        vst = pl.multiple_of(vbase + t * (CH // 8), 8)
        pltpu.sync_copy(gidx.at[pl.ds(base, CH)], gvs[b])
        pltpu.sync_copy(sidx.at[pl.ds(base, CH)], svs[b])
        pltpu.sync_copy(vrep.at[pl.ds(vst, CH // 8)], vvs[b])
        pltpu.async_copy(src.at[gvs[b]], rbs[b], sems[b])

    def finish(b):
        pltpu.make_async_copy(src.at[gvs[b]], rbs[b], sems[b]).wait()
        rbuf, vv = rbs[b], vvs[b]

        def rowgrp(io, _):
            rbase = io * 8
            for ii in range(8):
                val = vv[io, pl.ds(ii * 16, 16)]
                for cc in range(D // 16):
                    sl = pl.ds(cc * 16, 16)
                    rbuf[rbase + ii, sl] = rbuf[rbase + ii, sl] * val
            return _

        lax.fori_loop(0, CH // 8, rowgrp, None)
        pltpu.sync_copy(rbuf, acc.at[svs[b]], add=True)

    issue(0, 0)

    def pair(j, _):
        t0 = 2 * j

        @pl.when(t0 + 1 < nchunks)
        def _():
            issue(t0 + 1, 1)

        finish(0)

        @pl.when(t0 + 1 < nchunks)
        def _():
            @pl.when(t0 + 2 < nchunks)
            def _():
                issue(t0 + 2, 0)

            finish(1)

        return _

    lax.fori_loop(0, (nchunks + 1) // 2, pair, None)
    plsc.subcore_barrier()
    pltpu.sync_copy(acc.at[pl.ds(s * rows_per_sub, rows_per_sub)],
                    outp.at[c, pl.ds(s * rows_per_sub, rows_per_sub)])


def _sc_spmm(src, gidx, sidx, vrep, nout, nnz_pad):
    nchunks = nnz_pad // (NTILES * CH)
    fn = pl.kernel(
        functools.partial(_spmm_body, nchunks, nout),
        out_type=jax.ShapeDtypeStruct((2, nout, D), jnp.float32),
        mesh=_mesh,
        scratch_types=[
            pltpu.VMEM_SHARED((nout, D), jnp.float32),
            pltpu.VMEM((CH,), jnp.int32),
            pltpu.VMEM((CH,), jnp.int32),
            pltpu.VMEM((CH,), jnp.int32),
            pltpu.VMEM((CH,), jnp.int32),
            pltpu.VMEM((CH // 8, D), jnp.float32),
            pltpu.VMEM((CH // 8, D), jnp.float32),
            pltpu.VMEM((CH, D), jnp.float32),
            pltpu.VMEM((CH, D), jnp.float32),
            pltpu.SemaphoreType.DMA,
            pltpu.SemaphoreType.DMA,
            pltpu.SemaphoreType.DMA,
            pltpu.SemaphoreType.DMA,
        ],
    )
    return fn(src, gidx, sidx, vrep)


def _mm_body(h_ref, w_ref, o_ref):
    o_ref[...] = jnp.dot(h_ref[...], w_ref[...],
                         preferred_element_type=jnp.float32)


def _tc_matmul(h, w):
    n = h.shape[0]
    blk = 1024
    return pl.pallas_call(
        _mm_body,
        grid=(n // blk,),
        in_specs=[
            pl.BlockSpec((blk, D), lambda i: (i, 0)),
            pl.BlockSpec((D, D), lambda i: (0, 0)),
        ],
        out_specs=pl.BlockSpec((blk, D), lambda i: (i, 0)),
        out_shape=jax.ShapeDtypeStruct((n, D), jnp.float32),
    )(h, w)


def _comb_deg_body(relu, p_ref, c_ref, b_ref, o_ref):
    cnt = c_ref[0] + c_ref[1]
    deg = jnp.where(cnt > 0.0, 1.0 / jnp.maximum(cnt, 1.0), 0.0)
    v = (p_ref[0] + p_ref[1]) * deg + b_ref[...]
    if relu:
        v = jnp.maximum(v, 0.0)
    o_ref[...] = v


def _tc_comb_deg(partials, cnts, bias, relu):
    n = partials.shape[1]
    return pl.pallas_call(
        functools.partial(_comb_deg_body, relu),
        out_shape=jax.ShapeDtypeStruct((n, D), jnp.float32),
    )(partials, cnts, bias)


def _add_body(p_ref, o_ref):
    o_ref[...] = p_ref[0] + p_ref[1]


def _tc_add(partials):
    n = partials.shape[1]
    return pl.pallas_call(
        _add_body,
        out_shape=jax.ShapeDtypeStruct((n, D), jnp.float32),
    )(partials)


def _pool_body(h0_ref, h1_ref, tf_ref, gid_ref, mp0_ref, mp1_ref,
               awh0_ref, awt0_ref, ab0_ref, awh1_ref, awt1_ref, ab1_ref,
               lw0_ref, lb0_ref, lw1_ref, lb1_ref, o_ref):
    gid = gid_ref[...]
    iota = lax.broadcasted_iota(jnp.int32, (NG, NUM_V), 0)
    onehot = (jnp.broadcast_to(gid, (NG, NUM_V)) == iota).astype(jnp.float32)
    tf = tf_ref[...]

    def layer(h, mp_ref, awh_ref, awt_ref, ab_ref, lw_ref, lb_ref):
        elem = (lax.dot_general(h, awh_ref[...], (((1,), (0,)), ((), ())),
                                preferred_element_type=jnp.float32)
                + lax.dot_general(tf, awt_ref[...], (((1,), (0,)), ((), ())),
                                  preferred_element_type=jnp.float32)
                + ab_ref[...])
        mx = jnp.max(elem)
        e = jnp.exp(elem - mx)
        rs = lax.dot_general(onehot, e, (((1,), (0,)), ((), ())),
                             preferred_element_type=jnp.float32)
        pooled = lax.dot_general(onehot, h * e, (((1,), (0,)), ((), ())),
                                 preferred_element_type=jnp.float32)
        pooled = pooled / (rs + 1e-10)
        mpool = jnp.max(mp_ref[...], axis=1)
        ph = jnp.concatenate([pooled, mpool], axis=1)
        return (lax.dot_general(ph, lw_ref[...], (((1,), (0,)), ((), ())),
                                preferred_element_type=jnp.float32)
                + lb_ref[...])

    o_ref[...] = (layer(h0_ref[...], mp0_ref, awh0_ref, awt0_ref, ab0_ref,
                        lw0_ref, lb0_ref)
                  + layer(h1_ref[...], mp1_ref, awh1_ref, awt1_ref, ab1_ref,
                          lw1_ref, lb1_ref))


def _tc_pool(h0, h1, tfp, gid2d, mp0, mp1, awh0, awt0, ab0, awh1, awt1, ab1,
             lw0p, lb0p, lw1p, lb1p):
    return pl.pallas_call(
        _pool_body,
        out_shape=jax.ShapeDtypeStruct((NG, D), jnp.float32),
    )(h0, h1, tfp, gid2d, mp0, mp1, awh0, awt0, ab0, awh1, awt1, ab1,
      lw0p, lb0p, lw1p, lb1p)


def _pad1(a, n, fill):
    return jnp.concatenate(
        [a, jnp.full((n - a.shape[0],), fill, a.dtype)])


def kernel(x, inc_rows, inc_cols, inc_vals, sent_rows, sent_cols, sent_vals,
           graph_ids, max_pool_idx, tf_idf, emb, W_hg, b_hg,
           att_w0, att_b0, att_w1, att_b1, lin_w0, lin_b0, lin_w1, lin_b1):
    i32 = jnp.int32
    f32 = jnp.float32

    xp = _pad1(x.astype(i32), EMB_P, 0)
    ir = _pad1(inc_rows.astype(i32), NNZ_P, NV_P - 8)
    ic = _pad1(inc_cols.astype(i32), NNZ_P, NE_P - 8)
    iv = _pad1(inc_vals.astype(f32), NNZ_P, 0.0)
    ivrep = jnp.broadcast_to(iv[:, None], (NNZ_P, 16)).reshape(NNZ_P // 8, D)
    sr = _pad1(sent_rows.astype(i32), SENT_P, NE_P - 8)
    sc_ = _pad1(sent_cols.astype(i32), SENT_P, NE_P - 8)
    sv = _pad1(sent_vals.astype(f32), SENT_P, 0.0)
    svrep = jnp.broadcast_to(sv[:, None], (SENT_P, 16)).reshape(SENT_P // 8, D)

    h_full, cntv, cnte = _sc_k1(emb.astype(f32), xp, ir, ic)

    m = _tc_matmul(h_full[:NV_P], W_hg.astype(f32))

    edge_p = _sc_spmm(m, ir, ic, ivrep, NE_P, NNZ_P)
    zero_b = jnp.zeros((1, D), f32)
    edge = _tc_comb_deg(edge_p, cnte, zero_b, relu=False)

    e2_p = _sc_spmm(edge, sc_, sr, svrep, NE_P, SENT_P)
    edge2 = _tc_add(e2_p)

    node_p = _sc_spmm(edge2, ic, ir, ivrep, NV_P, NNZ_P)
    h2_full = _tc_comb_deg(node_p, cntv, b_hg.astype(f32).reshape(1, D),
                           relu=True)

    mpi = _pad1(max_pool_idx.reshape(-1).astype(i32), MP_P, 0)
    mp0_, mp1_ = _sc_mp(h_full, h2_full, mpi)
    mp0 = mp0_.reshape(NG, MAXN, D)
    mp1 = mp1_.reshape(NG, MAXN, D)

    tfp = jnp.pad(tf_idf.astype(f32), ((0, 0), (0, D - 2)))
    gid2d = graph_ids.astype(i32).reshape(1, NUM_V)
    awh0 = att_w0[:D].astype(f32)
    awt0 = jnp.pad(att_w0[D:D + 2].astype(f32), ((0, D - 2), (0, 0)))
    awh1 = att_w1[:D].astype(f32)
    awt1 = jnp.pad(att_w1[D:D + 2].astype(f32), ((0, D - 2), (0, 0)))
    lw0p = jnp.pad(lin_w0.astype(f32), ((0, 0), (0, D - lin_w0.shape[1])))
    lb0p = jnp.pad(lin_b0.astype(f32), (0, D - lin_b0.shape[0])).reshape(1, D)
    lw1p = jnp.pad(lin_w1.astype(f32), ((0, 0), (0, D - lin_w1.shape[1])))
    lb1p = jnp.pad(lin_b1.astype(f32), (0, D - lin_b1.shape[0])).reshape(1, D)
    ab0 = att_b0.astype(f32).reshape(1, 1)
    ab1 = att_b1.astype(f32).reshape(1, 1)

    pred = _tc_pool(h_full[:NUM_V], h2_full[:NUM_V], tfp, gid2d, mp0, mp1,
                    awh0, awt0, ab0, awh1, awt1, ab1, lw0p, lb0p, lw1p, lb1p)
    return pred[:, :lin_w0.shape[1]]


# speedup vs baseline: 1.0084x; 1.0084x over previous
import functools

import jax
import jax.numpy as jnp
from jax import lax
from jax.experimental import pallas as pl
from jax.experimental.pallas import tpu as pltpu
from jax.experimental.pallas import tpu_sc as plsc

NUM_V = 10000
NUM_E = 5000
NG = 16
MAXN = 1024
D = 128

NV_P = 10240
NE_P = 5120
NNZ_P = 327680
SENT_P = 16384
EMB_P = 12288
MP_P = 16384

CH = 128
NTILES = 32

_mesh = plsc.VectorSubcoreMesh(core_axis_name="c", subcore_axis_name="s")


def _wid():
    return lax.axis_index("c") * 16 + lax.axis_index("s")


def _gather_body(nchunks, src, idx, out, iv0, iv1, rb0, rb1, sem0, sem1):
    w = _wid()
    tbase = w * (nchunks * CH)
    ivs, rbs, sems = (iv0, iv1), (rb0, rb1), (sem0, sem1)

    def issue(t, b):
        pltpu.sync_copy(idx.at[pl.ds(tbase + t * CH, CH)], ivs[b])
        pltpu.async_copy(src.at[ivs[b]], rbs[b], sems[b])

    def finish(t, b):
        pltpu.make_async_copy(src.at[ivs[b]], rbs[b], sems[b]).wait()
        pltpu.sync_copy(rbs[b], out.at[pl.ds(tbase + t * CH, CH)])

    issue(0, 0)

    def pair(j, _):
        t0 = 2 * j

        @pl.when(t0 + 1 < nchunks)
        def _():
            issue(t0 + 1, 1)

        finish(t0, 0)

        @pl.when(t0 + 1 < nchunks)
        def _():
            @pl.when(t0 + 2 < nchunks)
            def _():
                issue(t0 + 2, 0)

            finish(t0 + 1, 1)

        return _

    lax.fori_loop(0, (nchunks + 1) // 2, pair, None)


def _sc_gather(src, idx, n_pad):
    nchunks = n_pad // (NTILES * CH)
    fn = pl.kernel(
        functools.partial(_gather_body, nchunks),
        out_type=jax.ShapeDtypeStruct((n_pad, D), jnp.float32),
        mesh=_mesh,
        scratch_types=[
            pltpu.VMEM((CH,), jnp.int32),
            pltpu.VMEM((CH,), jnp.int32),
            pltpu.VMEM((CH, D), jnp.float32),
            pltpu.VMEM((CH, D), jnp.float32),
            pltpu.SemaphoreType.DMA,
            pltpu.SemaphoreType.DMA,
        ],
    )
    return fn(src, idx)


def _k1_body(c_nchunks, ridx, cidx, cv_out, ce_out,
             acc, iv0, iv1, iv2, iv3, rb0, rb1, semc):
    c = lax.axis_index("c")
    s = lax.axis_index("s")
    w = c * 16 + s
    ivs = (iv0, iv1, iv2, iv3)

    def bfill(r, _):
        for cc in range(D // 16):
            rb0[r, pl.ds(cc * 16, 16)] = jnp.zeros((16,), jnp.float32)
            rb1[r, pl.ds(cc * 16, 16)] = jnp.ones((16,), jnp.float32)
        return _

    lax.fori_loop(0, CH, bfill, None)
    zsrc = rb0.at[pl.ds(0, 16)]

    def zacc_v(j, _):
        pltpu.sync_copy(zsrc, acc.at[pl.ds(s * (NV_P // 16) + j * 16, 16)])
        return _

    lax.fori_loop(0, NV_P // 16 // 16, zacc_v, None)
    plsc.subcore_barrier()

    def count_pass(idx_hbm):
        def group(g, _):
            for b in range(4):
                base = w * (c_nchunks * CH) + (g * 4 + b) * CH
                pltpu.sync_copy(idx_hbm.at[pl.ds(base, CH)], ivs[b])
                pltpu.async_copy(rb1, acc.at[ivs[b]], semc, add=True)
            for b in range(4):
                pltpu.make_async_copy(rb1, acc.at[ivs[b]], semc).wait()
            return _

        lax.fori_loop(0, c_nchunks // 4, group, None)

    count_pass(ridx)
    plsc.subcore_barrier()
    rv = NV_P // 16
    pltpu.sync_copy(acc.at[pl.ds(s * rv, rv)], cv_out.at[c, pl.ds(s * rv, rv)])
    plsc.subcore_barrier()

    def zacc_e(j, _):
        pltpu.sync_copy(zsrc, acc.at[pl.ds(s * (NE_P // 16) + j * 16, 16)])
        return _

    lax.fori_loop(0, NE_P // 16 // 16, zacc_e, None)
    plsc.subcore_barrier()
    count_pass(cidx)
    plsc.subcore_barrier()
    re = NE_P // 16
    pltpu.sync_copy(acc.at[pl.ds(s * re, re)], ce_out.at[c, pl.ds(s * re, re)])


def _sc_counts2(ridx, cidx):
    c_nchunks = NNZ_P // (NTILES * CH)
    fn = pl.kernel(
        functools.partial(_k1_body, c_nchunks),
        out_type=(
            jax.ShapeDtypeStruct((2, NV_P, D), jnp.float32),
            jax.ShapeDtypeStruct((2, NE_P, D), jnp.float32),
        ),
        mesh=_mesh,
        scratch_types=[
            pltpu.VMEM_SHARED((NV_P, D), jnp.float32),
            pltpu.VMEM((CH,), jnp.int32),
            pltpu.VMEM((CH,), jnp.int32),
            pltpu.VMEM((CH,), jnp.int32),
            pltpu.VMEM((CH,), jnp.int32),
            pltpu.VMEM((CH, D), jnp.float32),
            pltpu.VMEM((CH, D), jnp.float32),
            pltpu.SemaphoreType.DMA,
        ],
    )
    return fn(ridx, cidx)


def _mp_body(nchunks, src0, src1, idx, out0, out1,
             iv0, iv1, ra0, ra1, rc0, rc1, sem0, sem1):
    w = _wid()
    tbase = w * (nchunks * CH)
    ivs, ras, rcs, sems = (iv0, iv1), (ra0, ra1), (rc0, rc1), (sem0, sem1)

    def issue(t, b):
        pltpu.sync_copy(idx.at[pl.ds(tbase + t * CH, CH)], ivs[b])
        pltpu.async_copy(src0.at[ivs[b]], ras[b], sems[b])
        pltpu.async_copy(src1.at[ivs[b]], rcs[b], sems[b])

    def finish(t, b):
        pltpu.make_async_copy(src0.at[ivs[b]], ras[b], sems[b]).wait()
        pltpu.make_async_copy(src1.at[ivs[b]], rcs[b], sems[b]).wait()
        pltpu.sync_copy(ras[b], out0.at[pl.ds(tbase + t * CH, CH)])
        pltpu.sync_copy(rcs[b], out1.at[pl.ds(tbase + t * CH, CH)])

    issue(0, 0)

    def pair(j, _):
        t0 = 2 * j

        @pl.when(t0 + 1 < nchunks)
        def _():
            issue(t0 + 1, 1)

        finish(t0, 0)

        @pl.when(t0 + 1 < nchunks)
        def _():
            @pl.when(t0 + 2 < nchunks)
            def _():
                issue(t0 + 2, 0)

            finish(t0 + 1, 1)

        return _

    lax.fori_loop(0, (nchunks + 1) // 2, pair, None)


def _sc_mp(src0, src1, idx):
    nchunks = MP_P // (NTILES * CH)
    fn = pl.kernel(
        functools.partial(_mp_body, nchunks),
        out_type=(
            jax.ShapeDtypeStruct((MP_P, D), jnp.float32),
            jax.ShapeDtypeStruct((MP_P, D), jnp.float32),
        ),
        mesh=_mesh,
        scratch_types=[
            pltpu.VMEM((CH,), jnp.int32),
            pltpu.VMEM((CH,), jnp.int32),
            pltpu.VMEM((CH, D), jnp.float32),
            pltpu.VMEM((CH, D), jnp.float32),
            pltpu.VMEM((CH, D), jnp.float32),
            pltpu.VMEM((CH, D), jnp.float32),
            pltpu.SemaphoreType.DMA,
            pltpu.SemaphoreType.DMA,
        ],
    )
    return fn(src0, src1, idx)


def _spmm_body(nchunks, nout, src, gidx, sidx, vrep, outp,
               acc, gv0, gv1, sv0, sv1, vv0, vv1, rb0, rb1,
               sem0, sem1, sem2, sem3):
    c = lax.axis_index("c")
    s = lax.axis_index("s")
    w = c * 16 + s
    tbase = w * (nchunks * CH)
    gvs, svs, vvs = (gv0, gv1), (sv0, sv1), (vv0, vv1)
    rbs, sems = (rb0, rb1), (sem0, sem1, sem2, sem3)

    def zfill(r, _):
        for cc in range(D // 16):
            rb0[r, pl.ds(cc * 16, 16)] = jnp.zeros((16,), jnp.float32)
        return _

    lax.fori_loop(0, 16, zfill, None)
    rows_per_sub = nout // 16

    def zacc(j, _):
        pltpu.sync_copy(rb0.at[pl.ds(0, 16)],
                        acc.at[pl.ds(s * rows_per_sub + j * 16, 16)])
        return _

    lax.fori_loop(0, rows_per_sub // 16, zacc, None)
    plsc.subcore_barrier()

    vbase = w * (nchunks * (CH // 8))

    def issue(t, b):
        base = tbase + t * CH
        vst = pl.multiple_of(vbase + t * (CH // 8), 8)
        pltpu.sync_copy(gidx.at[pl.ds(base, CH)], gvs[b])
        pltpu.sync_copy(sidx.at[pl.ds(base, CH)], svs[b])
        pltpu.sync_copy(vrep.at[pl.ds(vst, CH // 8)], vvs[b])
        pltpu.async_copy(src.at[gvs[b]], rbs[b], sems[b])

    def finish(b):
        pltpu.make_async_copy(src.at[gvs[b]], rbs[b], sems[b]).wait()
        rbuf, vv = rbs[b], vvs[b]

        def rowgrp(io, _):
            rbase = io * 8
            for ii in range(8):
                val = vv[io, pl.ds(ii * 16, 16)]
                for cc in range(D // 16):
                    sl = pl.ds(cc * 16, 16)
                    rbuf[rbase + ii, sl] = rbuf[rbase + ii, sl] * val
            return _

        lax.fori_loop(0, CH // 8, rowgrp, None)
        pltpu.sync_copy(rbuf, acc.at[svs[b]], add=True)

    issue(0, 0)

    def pair(j, _):
        t0 = 2 * j

        @pl.when(t0 + 1 < nchunks)
        def _():
            issue(t0 + 1, 1)

        finish(0)

        @pl.when(t0 + 1 < nchunks)
        def _():
            @pl.when(t0 + 2 < nchunks)
            def _():
                issue(t0 + 2, 0)

            finish(1)

        return _

    lax.fori_loop(0, (nchunks + 1) // 2, pair, None)
    plsc.subcore_barrier()
    pltpu.sync_copy(acc.at[pl.ds(s * rows_per_sub, rows_per_sub)],
                    outp.at[c, pl.ds(s * rows_per_sub, rows_per_sub)])


def _sc_spmm(src, gidx, sidx, vrep, nout, nnz_pad):
    nchunks = nnz_pad // (NTILES * CH)
    fn = pl.kernel(
        functools.partial(_spmm_body, nchunks, nout),
        out_type=jax.ShapeDtypeStruct((2, nout, D), jnp.float32),
        mesh=_mesh,
        scratch_types=[
            pltpu.VMEM_SHARED((nout, D), jnp.float32),
            pltpu.VMEM((CH,), jnp.int32),
            pltpu.VMEM((CH,), jnp.int32),
            pltpu.VMEM((CH,), jnp.int32),
            pltpu.VMEM((CH,), jnp.int32),
            pltpu.VMEM((CH // 8, D), jnp.float32),
            pltpu.VMEM((CH // 8, D), jnp.float32),
            pltpu.VMEM((CH, D), jnp.float32),
            pltpu.VMEM((CH, D), jnp.float32),
            pltpu.SemaphoreType.DMA,
            pltpu.SemaphoreType.DMA,
            pltpu.SemaphoreType.DMA,
            pltpu.SemaphoreType.DMA,
        ],
    )
    return fn(src, gidx, sidx, vrep)


def _mm_body(h_ref, w_ref, o_ref):
    o_ref[...] = jnp.dot(h_ref[...], w_ref[...],
                         preferred_element_type=jnp.float32)


def _tc_matmul(h, w):
    n = h.shape[0]
    blk = 1024
    return pl.pallas_call(
        _mm_body,
        grid=(n // blk,),
        in_specs=[
            pl.BlockSpec((blk, D), lambda i: (i, 0)),
            pl.BlockSpec((D, D), lambda i: (0, 0)),
        ],
        out_specs=pl.BlockSpec((blk, D), lambda i: (i, 0)),
        out_shape=jax.ShapeDtypeStruct((n, D), jnp.float32),
    )(h, w)


def _comb_deg_body(relu, p_ref, c_ref, b_ref, o_ref):
    cnt = c_ref[0] + c_ref[1]
    deg = jnp.where(cnt > 0.0, 1.0 / jnp.maximum(cnt, 1.0), 0.0)
    v = (p_ref[0] + p_ref[1]) * deg + b_ref[...]
    if relu:
        v = jnp.maximum(v, 0.0)
    o_ref[...] = v


def _tc_comb_deg(partials, cnts, bias, relu):
    n = partials.shape[1]
    return pl.pallas_call(
        functools.partial(_comb_deg_body, relu),
        out_shape=jax.ShapeDtypeStruct((n, D), jnp.float32),
    )(partials, cnts, bias)


def _add_body(p_ref, o_ref):
    o_ref[...] = p_ref[0] + p_ref[1]


def _tc_add(partials):
    n = partials.shape[1]
    return pl.pallas_call(
        _add_body,
        out_shape=jax.ShapeDtypeStruct((n, D), jnp.float32),
    )(partials)


def _pool_body(h0_ref, h1_ref, tf_ref, gid_ref, mp0_ref, mp1_ref,
               awh0_ref, awt0_ref, ab0_ref, awh1_ref, awt1_ref, ab1_ref,
               lw0_ref, lb0_ref, lw1_ref, lb1_ref, o_ref):
    gid = gid_ref[...]
    iota = lax.broadcasted_iota(jnp.int32, (NG, NUM_V), 0)
    onehot = (jnp.broadcast_to(gid, (NG, NUM_V)) == iota).astype(jnp.float32)
    tf = tf_ref[...]

    def layer(h, mp_ref, awh_ref, awt_ref, ab_ref, lw_ref, lb_ref):
        elem = (lax.dot_general(h, awh_ref[...], (((1,), (0,)), ((), ())),
                                preferred_element_type=jnp.float32)
                + lax.dot_general(tf, awt_ref[...], (((1,), (0,)), ((), ())),
                                  preferred_element_type=jnp.float32)
                + ab_ref[...])
        mx = jnp.max(elem)
        e = jnp.exp(elem - mx)
        rs = lax.dot_general(onehot, e, (((1,), (0,)), ((), ())),
                             preferred_element_type=jnp.float32)
        pooled = lax.dot_general(onehot, h * e, (((1,), (0,)), ((), ())),
                                 preferred_element_type=jnp.float32)
        pooled = pooled / (rs + 1e-10)
        mpool = jnp.max(mp_ref[...], axis=1)
        ph = jnp.concatenate([pooled, mpool], axis=1)
        return (lax.dot_general(ph, lw_ref[...], (((1,), (0,)), ((), ())),
                                preferred_element_type=jnp.float32)
                + lb_ref[...])

    o_ref[...] = (layer(h0_ref[...], mp0_ref, awh0_ref, awt0_ref, ab0_ref,
                        lw0_ref, lb0_ref)
                  + layer(h1_ref[...], mp1_ref, awh1_ref, awt1_ref, ab1_ref,
                          lw1_ref, lb1_ref))


def _tc_pool(h0, h1, tfp, gid2d, mp0, mp1, awh0, awt0, ab0, awh1, awt1, ab1,
             lw0p, lb0p, lw1p, lb1p):
    return pl.pallas_call(
        _pool_body,
        out_shape=jax.ShapeDtypeStruct((NG, D), jnp.float32),
    )(h0, h1, tfp, gid2d, mp0, mp1, awh0, awt0, ab0, awh1, awt1, ab1,
      lw0p, lb0p, lw1p, lb1p)


def _pad1(a, n, fill):
    return jnp.concatenate(
        [a, jnp.full((n - a.shape[0],), fill, a.dtype)])


def kernel(x, inc_rows, inc_cols, inc_vals, sent_rows, sent_cols, sent_vals,
           graph_ids, max_pool_idx, tf_idf, emb, W_hg, b_hg,
           att_w0, att_b0, att_w1, att_b1, lin_w0, lin_b0, lin_w1, lin_b1):
    i32 = jnp.int32
    f32 = jnp.float32

    xp = _pad1(x.astype(i32), EMB_P, 0)
    ir = _pad1(inc_rows.astype(i32), NNZ_P, NV_P - 8)
    ic = _pad1(inc_cols.astype(i32), NNZ_P, NE_P - 8)
    iv = _pad1(inc_vals.astype(f32), NNZ_P, 0.0)
    ivrep = jnp.broadcast_to(iv[:, None], (NNZ_P, 16)).reshape(NNZ_P // 8, D)
    sr = _pad1(sent_rows.astype(i32), SENT_P, NE_P - 8)
    sc_ = _pad1(sent_cols.astype(i32), SENT_P, NE_P - 8)
    sv = _pad1(sent_vals.astype(f32), SENT_P, 0.0)
    svrep = jnp.broadcast_to(sv[:, None], (SENT_P, 16)).reshape(SENT_P // 8, D)

    h_full = _sc_gather(emb.astype(f32), xp, EMB_P)
    cntv, cnte = _sc_counts2(ir, ic)

    m = _tc_matmul(h_full[:NV_P], W_hg.astype(f32))

    edge_p = _sc_spmm(m, ir, ic, ivrep, NE_P, NNZ_P)
    zero_b = jnp.zeros((1, D), f32)
    edge = _tc_comb_deg(edge_p, cnte, zero_b, relu=False)

    e2_p = _sc_spmm(edge, sc_, sr, svrep, NE_P, SENT_P)
    edge2 = _tc_add(e2_p)

    node_p = _sc_spmm(edge2, ic, ir, ivrep, NV_P, NNZ_P)
    h2_full = _tc_comb_deg(node_p, cntv, b_hg.astype(f32).reshape(1, D),
                           relu=True)

    mpi = _pad1(max_pool_idx.reshape(-1).astype(i32), MP_P, 0)
    mp0_, mp1_ = _sc_mp(h_full, h2_full, mpi)
    mp0 = mp0_.reshape(NG, MAXN, D)
    mp1 = mp1_.reshape(NG, MAXN, D)

    tfp = jnp.pad(tf_idf.astype(f32), ((0, 0), (0, D - 2)))
    gid2d = graph_ids.astype(i32).reshape(1, NUM_V)
    awh0 = att_w0[:D].astype(f32)
    awt0 = jnp.pad(att_w0[D:D + 2].astype(f32), ((0, D - 2), (0, 0)))
    awh1 = att_w1[:D].astype(f32)
    awt1 = jnp.pad(att_w1[D:D + 2].astype(f32), ((0, D - 2), (0, 0)))
    lw0p = jnp.pad(lin_w0.astype(f32), ((0, 0), (0, D - lin_w0.shape[1])))
    lb0p = jnp.pad(lin_b0.astype(f32), (0, D - lin_b0.shape[0])).reshape(1, D)
    lw1p = jnp.pad(lin_w1.astype(f32), ((0, 0), (0, D - lin_w1.shape[1])))
    lb1p = jnp.pad(lin_b1.astype(f32), (0, D - lin_b1.shape[0])).reshape(1, D)
    ab0 = att_b0.astype(f32).reshape(1, 1)
    ab1 = att_b1.astype(f32).reshape(1, 1)

    pred = _tc_pool(h_full[:NUM_V], h2_full[:NUM_V], tfp, gid2d, mp0, mp1,
                    awh0, awt0, ab0, awh1, awt1, ab1, lw0p, lb0p, lw1p, lb1p)
    return pred[:, :lin_w0.shape[1]]


# speedup vs baseline: 1.1312x; 1.1218x over previous
import functools

import jax
import jax.numpy as jnp
from jax import lax
from jax.experimental import pallas as pl
from jax.experimental.pallas import tpu as pltpu
from jax.experimental.pallas import tpu_sc as plsc

NUM_V = 10000
NUM_E = 5000
NG = 16
MAXN = 1024
D = 128

NV_P = 10240
NE_P = 5120
NNZ_P = 327680
SENT_P = 16384
EMB_P = 12288
MP_P = 16384

CH = 128
NTILES = 32

_mesh = plsc.VectorSubcoreMesh(core_axis_name="c", subcore_axis_name="s")


def _wid():
    return lax.axis_index("c") * 16 + lax.axis_index("s")


def _gather_body(nchunks, src, idx, out, iv0, iv1, rb0, rb1, sem0, sem1):
    w = _wid()
    tbase = w * (nchunks * CH)
    ivs, rbs, sems = (iv0, iv1), (rb0, rb1), (sem0, sem1)

    def issue(t, b):
        pltpu.sync_copy(idx.at[pl.ds(tbase + t * CH, CH)], ivs[b])
        pltpu.async_copy(src.at[ivs[b]], rbs[b], sems[b])

    def finish(t, b):
        pltpu.make_async_copy(src.at[ivs[b]], rbs[b], sems[b]).wait()
        pltpu.sync_copy(rbs[b], out.at[pl.ds(tbase + t * CH, CH)])

    issue(0, 0)

    def pair(j, _):
        t0 = 2 * j

        @pl.when(t0 + 1 < nchunks)
        def _():
            issue(t0 + 1, 1)

        finish(t0, 0)

        @pl.when(t0 + 1 < nchunks)
        def _():
            @pl.when(t0 + 2 < nchunks)
            def _():
                issue(t0 + 2, 0)

            finish(t0 + 1, 1)

        return _

    lax.fori_loop(0, (nchunks + 1) // 2, pair, None)


def _sc_gather(src, idx, n_pad):
    nchunks = n_pad // (NTILES * CH)
    fn = pl.kernel(
        functools.partial(_gather_body, nchunks),
        out_type=jax.ShapeDtypeStruct((n_pad, D), jnp.float32),
        mesh=_mesh,
        scratch_types=[
            pltpu.VMEM((CH,), jnp.int32),
            pltpu.VMEM((CH,), jnp.int32),
            pltpu.VMEM((CH, D), jnp.float32),
            pltpu.VMEM((CH, D), jnp.float32),
            pltpu.SemaphoreType.DMA,
            pltpu.SemaphoreType.DMA,
        ],
    )
    return fn(src, idx)


def _count_body(nchunks, nbins, idx_hbm, out, acc, iv0, iv1, iv2, iv3,
                ones, zb, semc):
    c = lax.axis_index("c")
    s = lax.axis_index("s")
    w = c * 16 + s
    ivs = (iv0, iv1, iv2, iv3)

    def fill(r, _):
        for cc in range(D // 16):
            ones[r, pl.ds(cc * 16, 16)] = jnp.ones((16,), jnp.float32)
            zb[r % 16, pl.ds(cc * 16, 16)] = jnp.zeros((16,), jnp.float32)
        return _

    lax.fori_loop(0, CH, fill, None)
    rows_per_sub = nbins // 16

    def zacc(j, _):
        pltpu.sync_copy(zb, acc.at[pl.ds(s * rows_per_sub + j * 16, 16)])
        return _

    lax.fori_loop(0, rows_per_sub // 16, zacc, None)
    plsc.subcore_barrier()

    def group(g, _):
        for b in range(4):
            base = w * (nchunks * CH) + (g * 4 + b) * CH
            pltpu.sync_copy(idx_hbm.at[pl.ds(base, CH)], ivs[b])
            pltpu.async_copy(ones, acc.at[ivs[b]], semc, add=True)
        for b in range(4):
            pltpu.make_async_copy(ones, acc.at[ivs[b]], semc).wait()
        return _

    lax.fori_loop(0, nchunks // 4, group, None)
    plsc.subcore_barrier()
    pltpu.sync_copy(acc.at[pl.ds(s * rows_per_sub, rows_per_sub)],
                    out.at[c, pl.ds(s * rows_per_sub, rows_per_sub)])


def _sc_count_one(idx, nbins, nnz_pad):
    nchunks = nnz_pad // (NTILES * CH)
    fn = pl.kernel(
        functools.partial(_count_body, nchunks, nbins),
        out_type=jax.ShapeDtypeStruct((2, nbins, D), jnp.float32),
        mesh=_mesh,
        scratch_types=[
            pltpu.VMEM_SHARED((nbins, D), jnp.float32),
            pltpu.VMEM((CH,), jnp.int32),
            pltpu.VMEM((CH,), jnp.int32),
            pltpu.VMEM((CH,), jnp.int32),
            pltpu.VMEM((CH,), jnp.int32),
            pltpu.VMEM((CH, D), jnp.float32),
            pltpu.VMEM((16, D), jnp.float32),
            pltpu.SemaphoreType.DMA,
        ],
    )
    return fn(idx)


def _mp_body(nchunks, src0, src1, idx, out0, out1,
             iv0, iv1, ra0, ra1, rc0, rc1, sem0, sem1):
    w = _wid()
    tbase = w * (nchunks * CH)
    ivs, ras, rcs, sems = (iv0, iv1), (ra0, ra1), (rc0, rc1), (sem0, sem1)

    def issue(t, b):
        pltpu.sync_copy(idx.at[pl.ds(tbase + t * CH, CH)], ivs[b])
        pltpu.async_copy(src0.at[ivs[b]], ras[b], sems[b])
        pltpu.async_copy(src1.at[ivs[b]], rcs[b], sems[b])

    def finish(t, b):
        pltpu.make_async_copy(src0.at[ivs[b]], ras[b], sems[b]).wait()
        pltpu.make_async_copy(src1.at[ivs[b]], rcs[b], sems[b]).wait()
        pltpu.sync_copy(ras[b], out0.at[pl.ds(tbase + t * CH, CH)])
        pltpu.sync_copy(rcs[b], out1.at[pl.ds(tbase + t * CH, CH)])

    issue(0, 0)

    def pair(j, _):
        t0 = 2 * j

        @pl.when(t0 + 1 < nchunks)
        def _():
            issue(t0 + 1, 1)

        finish(t0, 0)

        @pl.when(t0 + 1 < nchunks)
        def _():
            @pl.when(t0 + 2 < nchunks)
            def _():
                issue(t0 + 2, 0)

            finish(t0 + 1, 1)

        return _

    lax.fori_loop(0, (nchunks + 1) // 2, pair, None)


def _sc_mp(src0, src1, idx):
    nchunks = MP_P // (NTILES * CH)
    fn = pl.kernel(
        functools.partial(_mp_body, nchunks),
        out_type=(
            jax.ShapeDtypeStruct((MP_P, D), jnp.float32),
            jax.ShapeDtypeStruct((MP_P, D), jnp.float32),
        ),
        mesh=_mesh,
        scratch_types=[
            pltpu.VMEM((CH,), jnp.int32),
            pltpu.VMEM((CH,), jnp.int32),
            pltpu.VMEM((CH, D), jnp.float32),
            pltpu.VMEM((CH, D), jnp.float32),
            pltpu.VMEM((CH, D), jnp.float32),
            pltpu.VMEM((CH, D), jnp.float32),
            pltpu.SemaphoreType.DMA,
            pltpu.SemaphoreType.DMA,
        ],
    )
    return fn(src0, src1, idx)


def _spmm_body(nchunks, nout, src, gidx, sidx, vrep, outp,
               acc, gv0, gv1, sv0, sv1, vv0, vv1, rb0, rb1,
               sem0, sem1, sem2, sem3):
    c = lax.axis_index("c")
    s = lax.axis_index("s")
    w = c * 16 + s
    tbase = w * (nchunks * CH)
    gvs, svs, vvs = (gv0, gv1), (sv0, sv1), (vv0, vv1)
    rbs, sems = (rb0, rb1), (sem0, sem1, sem2, sem3)

    def zfill(r, _):
        for cc in range(D // 16):
            rb0[r, pl.ds(cc * 16, 16)] = jnp.zeros((16,), jnp.float32)
        return _

    lax.fori_loop(0, 16, zfill, None)
    rows_per_sub = nout // 16

    def zacc(j, _):
        pltpu.sync_copy(rb0.at[pl.ds(0, 16)],
                        acc.at[pl.ds(s * rows_per_sub + j * 16, 16)])
        return _

    lax.fori_loop(0, rows_per_sub // 16, zacc, None)
    plsc.subcore_barrier()

    vbase = w * (nchunks * (CH // 8))

    def issue(t, b):
        base = tbase + t * CH
        vst = pl.multiple_of(vbase + t * (CH // 8), 8)
        pltpu.sync_copy(gidx.at[pl.ds(base, CH)], gvs[b])
        pltpu.sync_copy(sidx.at[pl.ds(base, CH)], svs[b])
        pltpu.sync_copy(vrep.at[pl.ds(vst, CH // 8)], vvs[b])
        pltpu.async_copy(src.at[gvs[b]], rbs[b], sems[b])

    def finish(b):
        pltpu.make_async_copy(src.at[gvs[b]], rbs[b], sems[b]).wait()
        rbuf, vv = rbs[b], vvs[b]

        def rowgrp(io, _):
            rbase = io * 8
            for ii in range(8):
                val = vv[io, pl.ds(ii * 16, 16)]
                for cc in range(D // 16):
                    sl = pl.ds(cc * 16, 16)
                    rbuf[rbase + ii, sl] = rbuf[rbase + ii, sl] * val
            return _

        lax.fori_loop(0, CH // 8, rowgrp, None)
        pltpu.sync_copy(rbuf, acc.at[svs[b]], add=True)

    issue(0, 0)

    def pair(j, _):
        t0 = 2 * j

        @pl.when(t0 + 1 < nchunks)
        def _():
            issue(t0 + 1, 1)

        finish(0)

        @pl.when(t0 + 1 < nchunks)
        def _():
            @pl.when(t0 + 2 < nchunks)
            def _():
                issue(t0 + 2, 0)

            finish(1)

        return _

    lax.fori_loop(0, (nchunks + 1) // 2, pair, None)
    plsc.subcore_barrier()
    pltpu.sync_copy(acc.at[pl.ds(s * rows_per_sub, rows_per_sub)],
                    outp.at[c, pl.ds(s * rows_per_sub, rows_per_sub)])


def _sc_spmm(src, gidx, sidx, vrep, nout, nnz_pad):
    nchunks = nnz_pad // (NTILES * CH)
    fn = pl.kernel(
        functools.partial(_spmm_body, nchunks, nout),
        out_type=jax.ShapeDtypeStruct((2, nout, D), jnp.float32),
        mesh=_mesh,
        scratch_types=[
            pltpu.VMEM_SHARED((nout, D), jnp.float32),
            pltpu.VMEM((CH,), jnp.int32),
            pltpu.VMEM((CH,), jnp.int32),
            pltpu.VMEM((CH,), jnp.int32),
            pltpu.VMEM((CH,), jnp.int32),
            pltpu.VMEM((CH // 8, D), jnp.float32),
            pltpu.VMEM((CH // 8, D), jnp.float32),
            pltpu.VMEM((CH, D), jnp.float32),
            pltpu.VMEM((CH, D), jnp.float32),
            pltpu.SemaphoreType.DMA,
            pltpu.SemaphoreType.DMA,
            pltpu.SemaphoreType.DMA,
            pltpu.SemaphoreType.DMA,
        ],
    )
    return fn(src, gidx, sidx, vrep)


def _mm_body(h_ref, w_ref, o_ref):
    o_ref[...] = jnp.dot(h_ref[...], w_ref[...],
                         preferred_element_type=jnp.float32)


def _tc_matmul(h, w):
    n = h.shape[0]
    blk = 1024
    return pl.pallas_call(
        _mm_body,
        grid=(n // blk,),
        in_specs=[
            pl.BlockSpec((blk, D), lambda i: (i, 0)),
            pl.BlockSpec((D, D), lambda i: (0, 0)),
        ],
        out_specs=pl.BlockSpec((blk, D), lambda i: (i, 0)),
        out_shape=jax.ShapeDtypeStruct((n, D), jnp.float32),
    )(h, w)


def _comb_deg_body(relu, p_ref, c_ref, b_ref, o_ref):
    cnt = c_ref[0] + c_ref[1]
    deg = jnp.where(cnt > 0.0, 1.0 / jnp.maximum(cnt, 1.0), 0.0)
    v = (p_ref[0] + p_ref[1]) * deg + b_ref[...]
    if relu:
        v = jnp.maximum(v, 0.0)
    o_ref[...] = v


def _tc_comb_deg(partials, cnts, bias, relu):
    n = partials.shape[1]
    return pl.pallas_call(
        functools.partial(_comb_deg_body, relu),
        out_shape=jax.ShapeDtypeStruct((n, D), jnp.float32),
    )(partials, cnts, bias)


def _add_body(p_ref, o_ref):
    o_ref[...] = p_ref[0] + p_ref[1]


def _tc_add(partials):
    n = partials.shape[1]
    return pl.pallas_call(
        _add_body,
        out_shape=jax.ShapeDtypeStruct((n, D), jnp.float32),
    )(partials)


def _pool_body(h0_ref, h1_ref, tf_ref, gid_ref, mp0_ref, mp1_ref,
               awh0_ref, awt0_ref, ab0_ref, awh1_ref, awt1_ref, ab1_ref,
               lw0_ref, lb0_ref, lw1_ref, lb1_ref, o_ref):
    gid = gid_ref[...]
    iota = lax.broadcasted_iota(jnp.int32, (NG, NUM_V), 0)
    onehot = (jnp.broadcast_to(gid, (NG, NUM_V)) == iota).astype(jnp.float32)
    tf = tf_ref[...]

    def layer(h, mp_ref, awh_ref, awt_ref, ab_ref, lw_ref, lb_ref):
        elem = (lax.dot_general(h, awh_ref[...], (((1,), (0,)), ((), ())),
                                preferred_element_type=jnp.float32)
                + lax.dot_general(tf, awt_ref[...], (((1,), (0,)), ((), ())),
                                  preferred_element_type=jnp.float32)
                + ab_ref[...])
        mx = jnp.max(elem)
        e = jnp.exp(elem - mx)
        rs = lax.dot_general(onehot, e, (((1,), (0,)), ((), ())),
                             preferred_element_type=jnp.float32)
        pooled = lax.dot_general(onehot, h * e, (((1,), (0,)), ((), ())),
                                 preferred_element_type=jnp.float32)
        pooled = pooled / (rs + 1e-10)
        mpool = jnp.max(mp_ref[...], axis=1)
        ph = jnp.concatenate([pooled, mpool], axis=1)
        return (lax.dot_general(ph, lw_ref[...], (((1,), (0,)), ((), ())),
                                preferred_element_type=jnp.float32)
                + lb_ref[...])

    o_ref[...] = (layer(h0_ref[...], mp0_ref, awh0_ref, awt0_ref, ab0_ref,
                        lw0_ref, lb0_ref)
                  + layer(h1_ref[...], mp1_ref, awh1_ref, awt1_ref, ab1_ref,
                          lw1_ref, lb1_ref))


def _tc_pool(h0, h1, tfp, gid2d, mp0, mp1, awh0, awt0, ab0, awh1, awt1, ab1,
             lw0p, lb0p, lw1p, lb1p):
    return pl.pallas_call(
        _pool_body,
        out_shape=jax.ShapeDtypeStruct((NG, D), jnp.float32),
    )(h0, h1, tfp, gid2d, mp0, mp1, awh0, awt0, ab0, awh1, awt1, ab1,
      lw0p, lb0p, lw1p, lb1p)


def _pad1(a, n, fill):
    return jnp.concatenate(
        [a, jnp.full((n - a.shape[0],), fill, a.dtype)])


def kernel(x, inc_rows, inc_cols, inc_vals, sent_rows, sent_cols, sent_vals,
           graph_ids, max_pool_idx, tf_idf, emb, W_hg, b_hg,
           att_w0, att_b0, att_w1, att_b1, lin_w0, lin_b0, lin_w1, lin_b1):
    i32 = jnp.int32
    f32 = jnp.float32

    xp = _pad1(x.astype(i32), EMB_P, 0)
    ir = _pad1(inc_rows.astype(i32), NNZ_P, NV_P - 8)
    ic = _pad1(inc_cols.astype(i32), NNZ_P, NE_P - 8)
    iv = _pad1(inc_vals.astype(f32), NNZ_P, 0.0)
    ivrep = jnp.broadcast_to(iv[:, None], (NNZ_P, 16)).reshape(NNZ_P // 8, D)
    sr = _pad1(sent_rows.astype(i32), SENT_P, NE_P - 8)
    sc_ = _pad1(sent_cols.astype(i32), SENT_P, NE_P - 8)
    sv = _pad1(sent_vals.astype(f32), SENT_P, 0.0)
    svrep = jnp.broadcast_to(sv[:, None], (SENT_P, 16)).reshape(SENT_P // 8, D)

    h_full = _sc_gather(emb.astype(f32), xp, EMB_P)
    cntv = _sc_count_one(ir, NV_P, NNZ_P)
    cnte = _sc_count_one(ic, NE_P, NNZ_P)

    m = _tc_matmul(h_full[:NV_P], W_hg.astype(f32))

    edge_p = _sc_spmm(m, ir, ic, ivrep, NE_P, NNZ_P)
    zero_b = jnp.zeros((1, D), f32)
    edge = _tc_comb_deg(edge_p, cnte, zero_b, relu=False)

    e2_p = _sc_spmm(edge, sc_, sr, svrep, NE_P, SENT_P)
    edge2 = _tc_add(e2_p)

    node_p = _sc_spmm(edge2, ic, ir, ivrep, NV_P, NNZ_P)
    h2_full = _tc_comb_deg(node_p, cntv, b_hg.astype(f32).reshape(1, D),
                           relu=True)

    mpi = _pad1(max_pool_idx.reshape(-1).astype(i32), MP_P, 0)
    mp0 = _sc_gather(h_full[:NV_P], mpi, MP_P).reshape(NG, MAXN, D)
    mp1 = _sc_gather(h2_full, mpi, MP_P).reshape(NG, MAXN, D)

    tfp = jnp.pad(tf_idf.astype(f32), ((0, 0), (0, D - 2)))
    gid2d = graph_ids.astype(i32).reshape(1, NUM_V)
    awh0 = att_w0[:D].astype(f32)
    awt0 = jnp.pad(att_w0[D:D + 2].astype(f32), ((0, D - 2), (0, 0)))
    awh1 = att_w1[:D].astype(f32)
    awt1 = jnp.pad(att_w1[D:D + 2].astype(f32), ((0, D - 2), (0, 0)))
    lw0p = jnp.pad(lin_w0.astype(f32), ((0, 0), (0, D - lin_w0.shape[1])))
    lb0p = jnp.pad(lin_b0.astype(f32), (0, D - lin_b0.shape[0])).reshape(1, D)
    lw1p = jnp.pad(lin_w1.astype(f32), ((0, 0), (0, D - lin_w1.shape[1])))
    lb1p = jnp.pad(lin_b1.astype(f32), (0, D - lin_b1.shape[0])).reshape(1, D)
    ab0 = att_b0.astype(f32).reshape(1, 1)
    ab1 = att_b1.astype(f32).reshape(1, 1)

    pred = _tc_pool(h_full[:NUM_V], h2_full[:NUM_V], tfp, gid2d, mp0, mp1,
                    awh0, awt0, ab0, awh1, awt1, ab1, lw0p, lb0p, lw1p, lb1p)
    return pred[:, :lin_w0.shape[1]]
